# baseline (speedup 1.0000x reference)
import jax
import jax.numpy as jnp
from jax import lax
from jax.experimental import pallas as pl
from jax.experimental.pallas import tpu as pltpu

D = 1024
E = 8
TOP_K = 2
TB = 1024

_DN_T = (((1,), (1,)), ((), ()))


def _moe_kernel(x_ref, Ws_ref, bs_ref, Wr_ref, br_ref, Wg_ref,
                bg_ref, gb_ref, out_ref, gates_ref):
    j = pl.program_id(1)
    x = x_ref[...]

    @pl.when(j == 0)
    def _gating_and_shared():
        scores = lax.dot_general(
            x, Wg_ref[...], _DN_T,
            preferred_element_type=jnp.float32) + bg_ref[...] + gb_ref[...]
        neg_inf = jnp.float32(-jnp.inf)
        v1 = jnp.max(scores, axis=-1, keepdims=True)
        eidx = lax.broadcasted_iota(jnp.int32, scores.shape, 1)
        a1 = jnp.min(jnp.where(scores == v1, eidx, E), axis=-1, keepdims=True)
        h1 = eidx == a1
        scores2 = jnp.where(h1, neg_inf, scores)
        v2 = jnp.max(scores2, axis=-1, keepdims=True)
        a2 = jnp.min(jnp.where(scores2 == v2, eidx, E), axis=-1, keepdims=True)
        h2 = eidx == a2
        w1 = jax.nn.sigmoid(v1 - v2)
        gates_ref[...] = h1 * w1 + h2 * (1.0 - w1)

        sh = lax.dot_general(x, Ws_ref[...], _DN_T,
                             preferred_element_type=jnp.float32)
        out_ref[...] = sh + bs_ref[...] + x

    @pl.when(j > 0)
    def _routed_expert():
        e = j - 1
        ye = lax.dot_general(x, Wr_ref[0], _DN_T,
                             preferred_element_type=jnp.float32)
        gates = gates_ref[...]
        eidx = lax.broadcasted_iota(jnp.int32, gates.shape, 1)
        g = jnp.sum(jnp.where(eidx == e, gates, 0.0), axis=1, keepdims=True)
        acc = out_ref[...] + g * (ye + br_ref[0])
        out_ref[...] = jnp.where(j == E, jnp.maximum(acc, 0.0), acc)


@jax.jit
def kernel(x, Ws, bs, Wr, br, Wg, bg, gate_bias):
    N = x.shape[0]
    bs2 = bs.reshape(1, D)
    br2 = br.reshape(E, 1, D)
    bg2 = bg.reshape(1, E)
    gb2 = gate_bias.reshape(1, E)

    grid = (N // TB, 1 + E)
    out = pl.pallas_call(
        _moe_kernel,
        grid=grid,
        in_specs=[
            pl.BlockSpec((TB, D), lambda i, j: (i, 0)),
            pl.BlockSpec((D, D), lambda i, j: (0, 0)),
            pl.BlockSpec((1, D), lambda i, j: (0, 0)),
            pl.BlockSpec((1, D, D), lambda i, j: (jnp.maximum(j - 1, 0), 0, 0)),
            pl.BlockSpec((1, 1, D), lambda i, j: (jnp.maximum(j - 1, 0), 0, 0)),
            pl.BlockSpec((E, D), lambda i, j: (0, 0)),
            pl.BlockSpec((1, E), lambda i, j: (0, 0)),
            pl.BlockSpec((1, E), lambda i, j: (0, 0)),
        ],
        out_specs=pl.BlockSpec((TB, D), lambda i, j: (i, 0)),
        out_shape=jax.ShapeDtypeStruct((N, D), jnp.float32),
        scratch_shapes=[pltpu.VMEM((TB, E), jnp.float32)],
        compiler_params=pltpu.CompilerParams(
            dimension_semantics=("arbitrary", "arbitrary")),
    )(x, Ws, bs2, Wr, br2, Wg, bg2, gb2)
    return out


# speedup vs baseline: 1.0634x; 1.0634x over previous
import jax
import jax.numpy as jnp
from jax import lax
from jax.experimental import pallas as pl
from jax.experimental.pallas import tpu as pltpu

D = 1024
E = 8
TOP_K = 2
TN = 512

_DN_T = (((1,), (1,)), ((), ()))


def _moe_kernel(x_ref, Ws_ref, bs_ref, Wr_hbm, br_ref, Wg_ref,
                bg_ref, gb_ref, out_ref, Wr_v, sems):
    i = pl.program_id(0)

    @pl.when(i == 0)
    def _start_weight_stream():
        for e in range(E):
            pltpu.make_async_copy(Wr_hbm.at[e], Wr_v.at[e], sems.at[e]).start()

    x = x_ref[...]

    scores = lax.dot_general(
        x, Wg_ref[...], _DN_T,
        preferred_element_type=jnp.float32) + bg_ref[...] + gb_ref[...]
    neg_inf = jnp.float32(-jnp.inf)
    v1 = jnp.max(scores, axis=-1, keepdims=True)
    eidx = lax.broadcasted_iota(jnp.int32, scores.shape, 1)
    a1 = jnp.min(jnp.where(scores == v1, eidx, E), axis=-1, keepdims=True)
    h1 = eidx == a1
    scores2 = jnp.where(h1, neg_inf, scores)
    v2 = jnp.max(scores2, axis=-1, keepdims=True)
    a2 = jnp.min(jnp.where(scores2 == v2, eidx, E), axis=-1, keepdims=True)
    h2 = eidx == a2
    w1 = jax.nn.sigmoid(v1 - v2)
    gates = h1 * w1 + h2 * (1.0 - w1)

    acc = lax.dot_general(x, Ws_ref[...], _DN_T,
                          preferred_element_type=jnp.float32) + bs_ref[...] + x

    for e in range(E):
        @pl.when(i == 0)
        def _wait_weight():
            pltpu.make_async_copy(Wr_hbm.at[e], Wr_v.at[e], sems.at[e]).wait()

        ye = lax.dot_general(x, Wr_v[e], _DN_T,
                             preferred_element_type=jnp.float32)
        acc = acc + gates[:, e:e + 1] * (ye + br_ref[e])

    out_ref[...] = jnp.maximum(acc, 0.0)


@jax.jit
def kernel(x, Ws, bs, Wr, br, Wg, bg, gate_bias):
    N = x.shape[0]
    bs2 = bs.reshape(1, D)
    br2 = br.reshape(E, 1, D)
    bg2 = bg.reshape(1, E)
    gb2 = gate_bias.reshape(1, E)

    grid = (N // TN,)
    out = pl.pallas_call(
        _moe_kernel,
        grid=grid,
        in_specs=[
            pl.BlockSpec((TN, D), lambda i: (i, 0)),
            pl.BlockSpec((D, D), lambda i: (0, 0)),
            pl.BlockSpec((1, D), lambda i: (0, 0)),
            pl.BlockSpec(memory_space=pl.ANY),
            pl.BlockSpec((E, 1, D), lambda i: (0, 0, 0)),
            pl.BlockSpec((E, D), lambda i: (0, 0)),
            pl.BlockSpec((1, E), lambda i: (0, 0)),
            pl.BlockSpec((1, E), lambda i: (0, 0)),
        ],
        out_specs=pl.BlockSpec((TN, D), lambda i: (i, 0)),
        out_shape=jax.ShapeDtypeStruct((N, D), jnp.float32),
        scratch_shapes=[
            pltpu.VMEM((E, D, D), jnp.float32),
            pltpu.SemaphoreType.DMA((E,)),
        ],
    )(x, Ws, bs2, Wr, br2, Wg, bg2, gb2)
    return out
